# baseline (speedup 1.0000x reference)
import functools

import jax
import jax.numpy as jnp
from jax import lax
from jax.experimental import pallas as pl
from jax.experimental.pallas import tpu as pltpu
from jax.experimental.pallas import tpu_sc as plsc

D = 64
B = 16384
NC = 2
NS = 16
NW = NC * NS
RW = B // NW
C = 128
NCHUNK = RW // C
MARGIN = 1.0

_mesh = plsc.VectorSubcoreMesh(core_axis_name="c", subcore_axis_name="s")


@functools.partial(
    pl.kernel,
    mesh=_mesh,
    compiler_params=pltpu.CompilerParams(
        needs_layout_passes=False, use_tc_tiling_on_sc=False),
    out_type=[
        jax.ShapeDtypeStruct((B,), jnp.float32),
        jax.ShapeDtypeStruct((B,), jnp.float32),
    ],
    scratch_types=[
        pltpu.VMEM((NCHUNK, C), jnp.int32),
        pltpu.VMEM((NCHUNK, C), jnp.int32),
        pltpu.VMEM((NCHUNK, C), jnp.int32),
        pltpu.VMEM((NCHUNK, C), jnp.int32),
        pltpu.VMEM((NCHUNK, C), jnp.int32),
        pltpu.VMEM((C, D), jnp.float32),
        pltpu.VMEM((C, D), jnp.float32),
        pltpu.VMEM((C, D), jnp.float32),
        pltpu.VMEM((C, D), jnp.float32),
        pltpu.VMEM((C, D), jnp.float32),
        pltpu.VMEM((RW,), jnp.float32),
        pltpu.VMEM((RW,), jnp.float32),
        pltpu.SemaphoreType.DMA,
    ],
)
def _sc_distances(ph_l, pt_l, nh_l, nt_l, r_l, e_emb, r_emb, out_p, out_n,
                  iph, ipt, inh, int_, ir,
                  bph, bpt, bnh, bnt, br,
                  op, on, sem):
    wid = lax.axis_index("s") * NC + lax.axis_index("c")
    base = wid * RW
    pltpu.sync_copy(ph_l.at[wid], iph)
    pltpu.sync_copy(pt_l.at[wid], ipt)
    pltpu.sync_copy(nh_l.at[wid], inh)
    pltpu.sync_copy(nt_l.at[wid], int_)
    pltpu.sync_copy(r_l.at[wid], ir)

    for j in range(NCHUNK):
        c1 = pltpu.async_copy(e_emb.at[iph.at[j]], bph, sem)
        c2 = pltpu.async_copy(e_emb.at[ipt.at[j]], bpt, sem)
        c3 = pltpu.async_copy(e_emb.at[inh.at[j]], bnh, sem)
        c4 = pltpu.async_copy(e_emb.at[int_.at[j]], bnt, sem)
        c5 = pltpu.async_copy(r_emb.at[ir.at[j]], br, sem)
        c1.wait(); c2.wait(); c3.wait(); c4.wait(); c5.wait()

        lanes = jnp.arange(16, dtype=jnp.int32)

        def group_body(g, _):
            resp = jnp.zeros((16,), jnp.float32)
            resn = jnp.zeros((16,), jnp.float32)
            for t in range(16):
                i = g * 16 + t
                accp = jnp.zeros((16,), jnp.float32)
                accn = jnp.zeros((16,), jnp.float32)
                for k in range(D // 16):
                    sl = pl.ds(k * 16, 16)
                    rv = br[i, sl]
                    dp = bph[i, sl] + rv - bpt[i, sl]
                    dn = bnh[i, sl] + rv - bnt[i, sl]
                    accp = accp + dp * dp
                    accn = accn + dn * dn
                resp = jnp.where(lanes == t, jnp.sum(accp), resp)
                resn = jnp.where(lanes == t, jnp.sum(accn), resn)
            op[pl.ds(j * C + g * 16, 16)] = resp
            on[pl.ds(j * C + g * 16, 16)] = resn
            return 0

        lax.fori_loop(0, C // 16, group_body, 0)

    pltpu.sync_copy(op, out_p.at[pl.ds(base, RW)])
    pltpu.sync_copy(on, out_n.at[pl.ds(base, RW)])


def _finish_body(p_ref, n_ref, o_ref):
    res = MARGIN + jnp.sqrt(p_ref[...]) - jnp.sqrt(n_ref[...])
    o_ref[...] = jnp.sum(jnp.where(res > 0, res, 0.0)).reshape(1, 1)


def kernel(posi_head_list, posi_tail_list, nege_head_list, nege_tail_list,
           r_list, e_embed, r_embed):
    ph3 = posi_head_list.reshape(NW, NCHUNK, C)
    pt3 = posi_tail_list.reshape(NW, NCHUNK, C)
    nh3 = nege_head_list.reshape(NW, NCHUNK, C)
    nt3 = nege_tail_list.reshape(NW, NCHUNK, C)
    r3 = r_list.reshape(NW, NCHUNK, C)
    psq, nsq = _sc_distances(ph3, pt3, nh3, nt3, r3, e_embed, r_embed)
    loss = pl.pallas_call(
        _finish_body,
        out_shape=jax.ShapeDtypeStruct((1, 1), jnp.float32),
    )(psq.reshape(128, 128), nsq.reshape(128, 128))
    return loss[0, 0]


# speedup vs baseline: 1.0998x; 1.0998x over previous
import functools

import jax
import jax.numpy as jnp
from jax import lax
from jax.experimental import pallas as pl
from jax.experimental.pallas import tpu as pltpu
from jax.experimental.pallas import tpu_sc as plsc

D = 64
DP = 128
B = 16384
NC = 2
NS = 16
NW = NC * NS
RW = B // NW
C = 128
NCHUNK = RW // C
MARGIN = 1.0

_mesh = plsc.VectorSubcoreMesh(core_axis_name="c", subcore_axis_name="s")


@functools.partial(
    pl.kernel,
    mesh=_mesh,
    compiler_params=pltpu.CompilerParams(
        needs_layout_passes=False, use_tc_tiling_on_sc=True),
    out_type=[
        jax.ShapeDtypeStruct((B,), jnp.float32),
        jax.ShapeDtypeStruct((B,), jnp.float32),
    ],
    scratch_types=[
        pltpu.VMEM((NCHUNK, C), jnp.int32),
        pltpu.VMEM((NCHUNK, C), jnp.int32),
        pltpu.VMEM((NCHUNK, C), jnp.int32),
        pltpu.VMEM((NCHUNK, C), jnp.int32),
        pltpu.VMEM((NCHUNK, C), jnp.int32),
        pltpu.VMEM((C, DP), jnp.float32),
        pltpu.VMEM((C, DP), jnp.float32),
        pltpu.VMEM((C, DP), jnp.float32),
        pltpu.VMEM((C, DP), jnp.float32),
        pltpu.VMEM((C, DP), jnp.float32),
        pltpu.VMEM((RW,), jnp.float32),
        pltpu.VMEM((RW,), jnp.float32),
        pltpu.SemaphoreType.DMA,
    ],
)
def _sc_distances(ph_l, pt_l, nh_l, nt_l, r_l, e_emb, r_emb, out_p, out_n,
                  iph, ipt, inh, int_, ir,
                  bph, bpt, bnh, bnt, br,
                  op, on, sem):
    wid = lax.axis_index("s") * NC + lax.axis_index("c")
    base = wid * RW
    pltpu.sync_copy(ph_l.at[wid], iph)
    pltpu.sync_copy(pt_l.at[wid], ipt)
    pltpu.sync_copy(nh_l.at[wid], inh)
    pltpu.sync_copy(nt_l.at[wid], int_)
    pltpu.sync_copy(r_l.at[wid], ir)

    for j in range(NCHUNK):
        c1 = pltpu.async_copy(e_emb.at[iph.at[j]], bph, sem)
        c2 = pltpu.async_copy(e_emb.at[ipt.at[j]], bpt, sem)
        c3 = pltpu.async_copy(e_emb.at[inh.at[j]], bnh, sem)
        c4 = pltpu.async_copy(e_emb.at[int_.at[j]], bnt, sem)
        c5 = pltpu.async_copy(r_emb.at[ir.at[j]], br, sem)
        c1.wait(); c2.wait(); c3.wait(); c4.wait(); c5.wait()

        lanes = jnp.arange(16, dtype=jnp.int32)

        def group_body(g, _):
            resp = jnp.zeros((16,), jnp.float32)
            resn = jnp.zeros((16,), jnp.float32)
            for t in range(16):
                i = g * 16 + t
                accp = jnp.zeros((16,), jnp.float32)
                accn = jnp.zeros((16,), jnp.float32)
                for k in range(D // 16):
                    sl = pl.ds(k * 16, 16)
                    rv = br[i, sl]
                    dp = bph[i, sl] + rv - bpt[i, sl]
                    dn = bnh[i, sl] + rv - bnt[i, sl]
                    accp = accp + dp * dp
                    accn = accn + dn * dn
                resp = jnp.where(lanes == t, jnp.sum(accp), resp)
                resn = jnp.where(lanes == t, jnp.sum(accn), resn)
            op[pl.ds(j * C + g * 16, 16)] = resp
            on[pl.ds(j * C + g * 16, 16)] = resn
            return 0

        lax.fori_loop(0, C // 16, group_body, 0)

    pltpu.sync_copy(op, out_p.at[pl.ds(base, RW)])
    pltpu.sync_copy(on, out_n.at[pl.ds(base, RW)])


def _finish_body(p_ref, n_ref, o_ref):
    res = MARGIN + jnp.sqrt(p_ref[...]) - jnp.sqrt(n_ref[...])
    o_ref[...] = jnp.sum(jnp.where(res > 0, res, 0.0)).reshape(1, 1)


def kernel(posi_head_list, posi_tail_list, nege_head_list, nege_tail_list,
           r_list, e_embed, r_embed):
    ph3 = posi_head_list.reshape(NW, NCHUNK, C)
    pt3 = posi_tail_list.reshape(NW, NCHUNK, C)
    nh3 = nege_head_list.reshape(NW, NCHUNK, C)
    nt3 = nege_tail_list.reshape(NW, NCHUNK, C)
    r3 = r_list.reshape(NW, NCHUNK, C)
    e_pad = jnp.pad(e_embed, ((0, 0), (0, DP - D)))
    r_pad = jnp.pad(r_embed, ((0, 0), (0, DP - D)))
    psq, nsq = _sc_distances(ph3, pt3, nh3, nt3, r3, e_pad, r_pad)
    loss = pl.pallas_call(
        _finish_body,
        out_shape=jax.ShapeDtypeStruct((1, 1), jnp.float32),
    )(psq.reshape(128, 128), nsq.reshape(128, 128))
    return loss[0, 0]
